# baseline (speedup 1.0000x reference)
import functools

import jax
import jax.numpy as jnp
from jax import lax
from jax.experimental import pallas as pl
from jax.experimental.pallas import tpu as pltpu
from jax.experimental.pallas import tpu_sc as plsc

N_UP = 2500
F = 128
K = 32
N_PAD = 10240
L = 16

RG = 8
FC = 4
RPG = N_PAD // RG
SUB = 5
RPS = RPG // SUB
NBLK = RPS // L
FCW = F // FC


def _body(feat_hbm, wt_hbm, nt_hbm, out_hbm, feat_c, idx_c, w_c, out_buf):
    wid = lax.axis_index("s") * 2 + lax.axis_index("c")
    rg = wid // FC
    fc = wid % FC
    row0g = rg * RPG
    col0 = fc * FCW
    pltpu.sync_copy(feat_hbm.at[pl.ds(col0, FCW)], feat_c)

    def sub(s, _):
        row0 = row0g + s * RPS
        pltpu.sync_copy(nt_hbm.at[:, pl.ds(row0, RPS)], idx_c)
        pltpu.sync_copy(wt_hbm.at[:, pl.ds(row0, RPS)], w_c)

        def block(b, _):
            rr = b * L
            for half in range(2):
                def kbody(k, accs):
                    vk = idx_c[k, pl.ds(rr, L)]
                    m = vk < N_UP
                    vkc = jnp.where(m, vk, 0)
                    wk = jnp.where(m, w_c[k, pl.ds(rr, L)], 0.0)
                    new = []
                    for f in range(L):
                        g = plsc.load_gather(feat_c.at[half * L + f], [vkc])
                        new.append(accs[f] + wk * g)
                    return tuple(new)

                accs = lax.fori_loop(
                    0, K, kbody,
                    tuple(jnp.zeros((L,), jnp.float32) for _ in range(L)),
                    unroll=4)
                for f in range(L):
                    out_buf[half * L + f, pl.ds(rr, L)] = accs[f] * (1.0 / K)
            return 0

        lax.fori_loop(0, NBLK, block, 0)
        pltpu.sync_copy(out_buf,
                        out_hbm.at[pl.ds(col0, FCW), pl.ds(row0, RPS)])
        return 0

    lax.fori_loop(0, SUB, sub, 0)


@jax.jit
def _sc_call(feat_t, wt, nt):
    mesh = plsc.VectorSubcoreMesh(core_axis_name="c", subcore_axis_name="s")
    return pl.kernel(
        _body,
        out_type=jax.ShapeDtypeStruct((F, N_PAD), jnp.float32),
        mesh=mesh,
        compiler_params=pltpu.CompilerParams(use_tc_tiling_on_sc=False,
                                             needs_layout_passes=False),
        scratch_types=[
            pltpu.VMEM((FCW, N_UP), jnp.float32),
            pltpu.VMEM((K, RPS), jnp.int32),
            pltpu.VMEM((K, RPS), jnp.float32),
            pltpu.VMEM((FCW, RPS), jnp.float32),
        ],
    )(feat_t, wt, nt)


def kernel(features, sel_idx_up, weights_down, nidx_down):
    n_down = weights_down.shape[0]
    pad = N_PAD - n_down
    wt = jnp.pad(weights_down, ((0, pad), (0, 0))).T
    nt = jnp.pad(nidx_down, ((0, pad), (0, 0))).T
    out_t = _sc_call(features.T, wt, nt)
    return out_t.T[:n_down]
